# Initial kernel scaffold; baseline (speedup 1.0000x reference)
import jax
import jax.numpy as jnp
from jax.experimental import pallas as pl


def kernel(x, edge_index, batch, W1_root, W1_nei, b1, W2_root, W2_nei, b2):
    raise NotImplementedError("write your pallas kernel here")


# speedup vs baseline: 4.7774x; 4.7774x over previous
import functools

import jax
import jax.numpy as jnp
from jax import lax
from jax.experimental import pallas as pl
from jax.experimental.pallas import tpu as pltpu
from jax.experimental.pallas import tpu_sc as plsc

_N = 10000
_E = 320000
_D = 128
_G = 64

_NC = 2
_NS = 16
_NW = _NC * _NS
_EPT = _E // _NW
_CHUNK = 80
_NCHUNK = _EPT // _CHUNK
_ZROWS = 16
_NZCH = _N // _ZROWS
_WROWS = 80
_NWCH = _N // _WROWS


def _segment_sum_sc(x, src, dst):
    d = x.shape[1]
    mesh = plsc.VectorSubcoreMesh(core_axis_name="c", subcore_axis_name="s")

    @functools.partial(
        pl.kernel,
        out_type=jax.ShapeDtypeStruct((_NC, _N, d), jnp.float32),
        mesh=mesh,
        scratch_types=[
            pltpu.VMEM((_CHUNK,), jnp.int32),
            pltpu.VMEM((_CHUNK,), jnp.int32),
            pltpu.VMEM((_CHUNK, d), jnp.float32),
            pltpu.VMEM((_ZROWS, d), jnp.float32),
            pltpu.VMEM_SHARED((_N, d), jnp.float32),
            pltpu.SemaphoreType.DMA,
        ],
    )
    def k(x_hbm, src_hbm, dst_hbm, out_hbm, sidx, didx, rows, zbuf, acc, sem):
        c = lax.axis_index("c")
        s = lax.axis_index("s")
        wid = c * _NS + s

        zero = jnp.zeros((16,), jnp.float32)
        for r in range(_ZROWS):
            for j in range(d // 16):
                zbuf[r, pl.ds(j * 16, 16)] = zero

        nz = jnp.where(s == 0, _NZCH // _NS + 1, _NZCH // _NS)

        def zloop(i, carry):
            pltpu.sync_copy(
                zbuf, acc.at[pl.ds((s + i * _NS) * _ZROWS, _ZROWS)])
            return carry

        lax.fori_loop(0, nz, zloop, 0)
        plsc.subcore_barrier()

        base = wid * _EPT

        def body(i, carry):
            off = base + i * _CHUNK
            pltpu.sync_copy(src_hbm.at[pl.ds(off, _CHUNK)], sidx)
            pltpu.sync_copy(dst_hbm.at[pl.ds(off, _CHUNK)], didx)
            pltpu.async_copy(x_hbm.at[sidx], rows, sem).wait()
            pltpu.sync_copy(rows, acc.at[didx], add=True)
            return carry

        lax.fori_loop(0, _NCHUNK, body, 0)
        plsc.subcore_barrier()

        nw = jnp.where(s < _NWCH - _NS * (_NWCH // _NS),
                       _NWCH // _NS + 1, _NWCH // _NS)

        def wloop(i, carry):
            r0 = (s + i * _NS) * _WROWS
            pltpu.sync_copy(
                acc.at[pl.ds(r0, _WROWS)],
                out_hbm.at[c, pl.ds(r0, _WROWS)],
            )
            return carry

        lax.fori_loop(0, nw, wloop, 0)

    return k(x, src, dst)


_BLK = 1000


def _layer1_tc(x, p, w_root, w_nei, b):
    d = x.shape[1]
    h = w_root.shape[1]

    def body(x_ref, p0_ref, p1_ref, wr_ref, wn_ref, b_ref, o_ref):
        agg = p0_ref[...] + p1_ref[...]
        acc = jnp.dot(x_ref[...], wr_ref[...], preferred_element_type=jnp.float32)
        acc = acc + jnp.dot(agg, wn_ref[...], preferred_element_type=jnp.float32)
        o_ref[...] = jnp.maximum(acc + b_ref[...], 0.0)

    return pl.pallas_call(
        body,
        grid=(_N // _BLK,),
        in_specs=[
            pl.BlockSpec((_BLK, d), lambda i: (i, 0)),
            pl.BlockSpec((_BLK, d), lambda i: (i, 0)),
            pl.BlockSpec((_BLK, d), lambda i: (i, 0)),
            pl.BlockSpec((d, h), lambda i: (0, 0)),
            pl.BlockSpec((d, h), lambda i: (0, 0)),
            pl.BlockSpec((1, h), lambda i: (0, 0)),
        ],
        out_specs=pl.BlockSpec((_BLK, h), lambda i: (i, 0)),
        out_shape=jax.ShapeDtypeStruct((_N, h), jnp.float32),
    )(x, p[0], p[1], w_root, w_nei, b.reshape(1, h))


def _layer2_pool_tc(x, p, w_root, w_nei, b, batch):
    d = x.shape[1]
    h = w_root.shape[1]
    nblk = _N // _BLK

    def body(x_ref, p0_ref, p1_ref, wr_ref, wn_ref, b_ref, bat_ref, o_ref,
             acc_ref, cnt_ref):
        i = pl.program_id(0)
        agg = p0_ref[...] + p1_ref[...]
        hh = jnp.dot(x_ref[...], wr_ref[...], preferred_element_type=jnp.float32)
        hh = hh + jnp.dot(agg, wn_ref[...], preferred_element_type=jnp.float32)
        hh = jnp.maximum(hh + b_ref[...], 0.0)

        onehot = (bat_ref[...] ==
                  lax.broadcasted_iota(jnp.int32, (_BLK, _G), 1)
                  ).astype(jnp.float32)
        part = lax.dot_general(onehot, hh, (((0,), (0,)), ((), ())),
                               preferred_element_type=jnp.float32)
        ones = jnp.ones((_BLK, h), jnp.float32)
        pcnt = lax.dot_general(onehot, ones, (((0,), (0,)), ((), ())),
                               preferred_element_type=jnp.float32)

        @pl.when(i == 0)
        def _():
            acc_ref[...] = jnp.zeros_like(acc_ref)
            cnt_ref[...] = jnp.zeros_like(cnt_ref)

        acc_ref[...] += part
        cnt_ref[...] += pcnt

        @pl.when(i == nblk - 1)
        def _():
            o_ref[...] = acc_ref[...] / jnp.maximum(cnt_ref[...], 1.0)

    return pl.pallas_call(
        body,
        grid=(nblk,),
        in_specs=[
            pl.BlockSpec((_BLK, d), lambda i: (i, 0)),
            pl.BlockSpec((_BLK, d), lambda i: (i, 0)),
            pl.BlockSpec((_BLK, d), lambda i: (i, 0)),
            pl.BlockSpec((d, h), lambda i: (0, 0)),
            pl.BlockSpec((d, h), lambda i: (0, 0)),
            pl.BlockSpec((1, h), lambda i: (0, 0)),
            pl.BlockSpec((_BLK, 1), lambda i: (i, 0)),
        ],
        out_specs=pl.BlockSpec((_G, h), lambda i: (0, 0)),
        out_shape=jax.ShapeDtypeStruct((_G, h), jnp.float32),
        scratch_shapes=[
            pltpu.VMEM((_G, h), jnp.float32),
            pltpu.VMEM((_G, h), jnp.float32),
        ],
    )(x, p[0], p[1], w_root, w_nei, b.reshape(1, h), batch.reshape(_N, 1))


def kernel(x, edge_index, batch, W1_root, W1_nei, b1, W2_root, W2_nei, b2):
    src = edge_index[0]
    dst = edge_index[1]
    p1 = _segment_sum_sc(x, src, dst)
    h = _layer1_tc(x, p1, W1_root, W1_nei, b1)
    p2 = _segment_sum_sc(h, src, dst)
    return _layer2_pool_tc(h, p2, W2_root, W2_nei, b2, batch)


# speedup vs baseline: 8.6231x; 1.8050x over previous
import functools

import jax
import jax.numpy as jnp
from jax import lax
from jax.experimental import pallas as pl
from jax.experimental.pallas import tpu as pltpu
from jax.experimental.pallas import tpu_sc as plsc

_N = 10000
_E = 320000
_D = 128
_G = 64

_NC = 2
_NS = 16
_NW = _NC * _NS
_EPT = _E // _NW
_CHUNK = 80
_NCHUNK = _EPT // _CHUNK
_WROWS = 80
_NWCH = _N // _WROWS


def _segment_sum_sc(x, src, dst):
    d = x.shape[1]
    mesh = plsc.VectorSubcoreMesh(core_axis_name="c", subcore_axis_name="s")

    @functools.partial(
        pl.kernel,
        out_type=jax.ShapeDtypeStruct((_NC, _N, d), jnp.float32),
        mesh=mesh,
        scratch_types=[
            pltpu.VMEM((_EPT,), jnp.int32),
            pltpu.VMEM((_NCHUNK, _CHUNK), jnp.int32),
            pltpu.VMEM((_CHUNK, d), jnp.float32),
            pltpu.VMEM((_CHUNK, d), jnp.float32),
            pltpu.VMEM_SHARED((_N, d), jnp.float32),
            pltpu.SemaphoreType.DMA,
            pltpu.SemaphoreType.DMA,
            pltpu.SemaphoreType.DMA,
            pltpu.SemaphoreType.DMA,
        ],
    )
    def k(x_hbm, src_hbm, dst_hbm, out_hbm, sidx, didx, rows_a, rows_b, acc,
          gsem_a, gsem_b, ssem_a, ssem_b):
        c = lax.axis_index("c")
        s = lax.axis_index("s")
        wid = c * _NS + s

        pltpu.sync_copy(src_hbm.at[wid], sidx)
        pltpu.sync_copy(dst_hbm.at[wid], didx)

        zero = jnp.zeros((16,), jnp.float32)

        def zrow(r, carry):
            for j in range(d // 16):
                rows_a[r, pl.ds(j * 16, 16)] = zero
            return carry

        lax.fori_loop(0, _CHUNK, zrow, 0)

        nz = jnp.where(s < _NWCH - _NS * (_NWCH // _NS),
                       _NWCH // _NS + 1, _NWCH // _NS)

        def zloop(i, carry):
            pltpu.sync_copy(
                rows_a, acc.at[pl.ds((s + i * _NS) * _CHUNK, _CHUNK)])
            return carry

        lax.fori_loop(0, nz, zloop, 0)
        plsc.subcore_barrier()

        def gather_start(j, rows, sem):
            pltpu.async_copy(
                x_hbm.at[sidx.at[pl.ds(j * _CHUNK, _CHUNK)]], rows, sem)

        def gather_wait(j, rows, sem):
            pltpu.make_async_copy(
                x_hbm.at[sidx.at[pl.ds(j * _CHUNK, _CHUNK)]], rows, sem).wait()

        def scatter_start(j, rows, sem):
            pltpu.async_copy(rows, acc.at[didx.at[j]], sem, add=True)

        def scatter_wait(j, rows, sem):
            pltpu.make_async_copy(rows, acc.at[didx.at[j]], sem).wait()

        gather_start(0, rows_a, gsem_a)

        def body(i, carry):
            j = 2 * i
            gather_wait(j, rows_a, gsem_a)
            gather_start(j + 1, rows_b, gsem_b)
            scatter_start(j, rows_a, ssem_a)
            gather_wait(j + 1, rows_b, gsem_b)
            scatter_wait(j, rows_a, ssem_a)
            gather_start(j + 2, rows_a, gsem_a)
            scatter_start(j + 1, rows_b, ssem_b)
            scatter_wait(j + 1, rows_b, ssem_b)
            return carry

        lax.fori_loop(0, (_NCHUNK - 1) // 2, body, 0)

        gather_wait(_NCHUNK - 1, rows_a, gsem_a)
        pltpu.sync_copy(rows_a, acc.at[didx.at[_NCHUNK - 1]], add=True)
        plsc.subcore_barrier()

        def wloop(i, carry):
            r0 = (s + i * _NS) * _WROWS
            pltpu.sync_copy(
                acc.at[pl.ds(r0, _WROWS)],
                out_hbm.at[c, pl.ds(r0, _WROWS)],
            )
            return carry

        lax.fori_loop(0, nz, wloop, 0)

    return k(x, src, dst)


_BLK = 1000


def _layer1_tc(x, p, w_root, w_nei, b):
    d = x.shape[1]
    h = w_root.shape[1]

    def body(x_ref, p0_ref, p1_ref, wr_ref, wn_ref, b_ref, o_ref):
        agg = p0_ref[...] + p1_ref[...]
        acc = jnp.dot(x_ref[...], wr_ref[...], preferred_element_type=jnp.float32)
        acc = acc + jnp.dot(agg, wn_ref[...], preferred_element_type=jnp.float32)
        o_ref[...] = jnp.maximum(acc + b_ref[...], 0.0)

    return pl.pallas_call(
        body,
        grid=(_N // _BLK,),
        in_specs=[
            pl.BlockSpec((_BLK, d), lambda i: (i, 0)),
            pl.BlockSpec((_BLK, d), lambda i: (i, 0)),
            pl.BlockSpec((_BLK, d), lambda i: (i, 0)),
            pl.BlockSpec((d, h), lambda i: (0, 0)),
            pl.BlockSpec((d, h), lambda i: (0, 0)),
            pl.BlockSpec((1, h), lambda i: (0, 0)),
        ],
        out_specs=pl.BlockSpec((_BLK, h), lambda i: (i, 0)),
        out_shape=jax.ShapeDtypeStruct((_N, h), jnp.float32),
    )(x, p[0], p[1], w_root, w_nei, b.reshape(1, h))


def _layer2_pool_tc(x, p, w_root, w_nei, b, batch):
    d = x.shape[1]
    h = w_root.shape[1]
    nblk = _N // _BLK

    def body(x_ref, p0_ref, p1_ref, wr_ref, wn_ref, b_ref, bat_ref, o_ref,
             acc_ref, cnt_ref):
        i = pl.program_id(0)
        agg = p0_ref[...] + p1_ref[...]
        hh = jnp.dot(x_ref[...], wr_ref[...], preferred_element_type=jnp.float32)
        hh = hh + jnp.dot(agg, wn_ref[...], preferred_element_type=jnp.float32)
        hh = jnp.maximum(hh + b_ref[...], 0.0)

        onehot = (bat_ref[...] ==
                  lax.broadcasted_iota(jnp.int32, (_BLK, _G), 1)
                  ).astype(jnp.float32)
        part = lax.dot_general(onehot, hh, (((0,), (0,)), ((), ())),
                               preferred_element_type=jnp.float32)
        ones = jnp.ones((_BLK, h), jnp.float32)
        pcnt = lax.dot_general(onehot, ones, (((0,), (0,)), ((), ())),
                               preferred_element_type=jnp.float32)

        @pl.when(i == 0)
        def _():
            acc_ref[...] = jnp.zeros_like(acc_ref)
            cnt_ref[...] = jnp.zeros_like(cnt_ref)

        acc_ref[...] += part
        cnt_ref[...] += pcnt

        @pl.when(i == nblk - 1)
        def _():
            o_ref[...] = acc_ref[...] / jnp.maximum(cnt_ref[...], 1.0)

    return pl.pallas_call(
        body,
        grid=(nblk,),
        in_specs=[
            pl.BlockSpec((_BLK, d), lambda i: (i, 0)),
            pl.BlockSpec((_BLK, d), lambda i: (i, 0)),
            pl.BlockSpec((_BLK, d), lambda i: (i, 0)),
            pl.BlockSpec((d, h), lambda i: (0, 0)),
            pl.BlockSpec((d, h), lambda i: (0, 0)),
            pl.BlockSpec((1, h), lambda i: (0, 0)),
            pl.BlockSpec((_BLK, 1), lambda i: (i, 0)),
        ],
        out_specs=pl.BlockSpec((_G, h), lambda i: (0, 0)),
        out_shape=jax.ShapeDtypeStruct((_G, h), jnp.float32),
        scratch_shapes=[
            pltpu.VMEM((_G, h), jnp.float32),
            pltpu.VMEM((_G, h), jnp.float32),
        ],
    )(x, p[0], p[1], w_root, w_nei, b.reshape(1, h), batch.reshape(_N, 1))


def kernel(x, edge_index, batch, W1_root, W1_nei, b1, W2_root, W2_nei, b2):
    src = edge_index[0].reshape(_NW, _EPT)
    dst = edge_index[1].reshape(_NW, _NCHUNK, _CHUNK)
    p1 = _segment_sum_sc(x, src, dst)
    h = _layer1_tc(x, p1, W1_root, W1_nei, b1)
    p2 = _segment_sum_sc(h, src, dst)
    return _layer2_pool_tc(h, p2, W2_root, W2_nei, b2, batch)


# speedup vs baseline: 11.9494x; 1.3857x over previous
import functools

import jax
import jax.numpy as jnp
from jax import lax
from jax.experimental import pallas as pl
from jax.experimental.pallas import tpu as pltpu
from jax.experimental.pallas import tpu_sc as plsc

_N = 10000
_E = 320000
_D = 128
_G = 64

_NC = 2
_NS = 16
_NW = _NC * _NS
_EPT = _E // _NW
_CHUNK = 80
_NCHUNK = _EPT // _CHUNK
_WROWS = 80
_NWCH = _N // _WROWS


_NBUF = 4


def _segment_sum_sc(x, src, dst):
    d = x.shape[1]
    mesh = plsc.VectorSubcoreMesh(core_axis_name="c", subcore_axis_name="s")

    @functools.partial(
        pl.kernel,
        out_type=jax.ShapeDtypeStruct((_NC, _N, d), jnp.float32),
        mesh=mesh,
        scratch_types=[
            pltpu.VMEM((_NBUF * _CHUNK,), jnp.int32),
            pltpu.VMEM((_NBUF * 8, _CHUNK), jnp.int32),
            [pltpu.VMEM((_CHUNK, d), jnp.float32) for _ in range(_NBUF)],
            pltpu.VMEM_SHARED((_N, d), jnp.float32),
            [pltpu.SemaphoreType.DMA for _ in range(3 * _NBUF)],
        ],
    )
    def k(x_hbm, src_hbm, dst_hbm, out_hbm, sidx, didx, rows, acc, sems):
        c = lax.axis_index("c")
        s = lax.axis_index("s")
        wid = c * _NS + s
        isem = sems[:_NBUF]
        gsem = sems[_NBUF:2 * _NBUF]
        ssem = sems[2 * _NBUF:]

        zero = jnp.zeros((16,), jnp.float32)

        def zrow(r, carry):
            for j in range(d // 16):
                rows[0][r, pl.ds(j * 16, 16)] = zero
            return carry

        lax.fori_loop(0, _CHUNK, zrow, 0)

        nz = jnp.where(s < _NWCH - _NS * (_NWCH // _NS),
                       _NWCH // _NS + 1, _NWCH // _NS)

        def zloop(i, carry):
            pltpu.sync_copy(
                rows[0], acc.at[pl.ds((s + i * _NS) * _CHUNK, _CHUNK)])
            return carry

        lax.fori_loop(0, nz, zloop, 0)
        plsc.subcore_barrier()

        def idx_start(j, b):
            pltpu.async_copy(
                src_hbm.at[wid, j, 0],
                sidx.at[pl.ds(b * _CHUNK, _CHUNK)], isem[b])
            pltpu.async_copy(dst_hbm.at[wid, j, 0], didx.at[8 * b], isem[b])

        def idx_wait(j, b):
            pltpu.make_async_copy(
                src_hbm.at[wid, j, 0],
                sidx.at[pl.ds(b * _CHUNK, _CHUNK)], isem[b]).wait()
            pltpu.make_async_copy(
                dst_hbm.at[wid, j, 0], didx.at[8 * b], isem[b]).wait()

        def gather_start(b):
            pltpu.async_copy(
                x_hbm.at[sidx.at[pl.ds(b * _CHUNK, _CHUNK)]], rows[b],
                gsem[b])

        def gather_wait(b):
            pltpu.make_async_copy(
                x_hbm.at[sidx.at[pl.ds(b * _CHUNK, _CHUNK)]], rows[b],
                gsem[b]).wait()

        def scatter_start(b):
            pltpu.async_copy(rows[b], acc.at[didx.at[8 * b]], ssem[b],
                             add=True)

        def scatter_wait(b):
            pltpu.make_async_copy(rows[b], acc.at[didx.at[8 * b]],
                                  ssem[b]).wait()

        def emit(j, b, swait_prev=True, idx_pf=True, g_pf=True):
            gather_wait(b)
            scatter_start(b)
            if swait_prev:
                scatter_wait((b + 3) % _NBUF)
            if idx_pf:
                idx_start(j + 3, (b + 3) % _NBUF)
            if g_pf:
                nb = (b + 2) % _NBUF
                idx_wait(j + 2, nb)
                gather_start(nb)

        idx_start(0, 0)
        idx_start(1, 1)
        idx_start(2, 2)
        idx_wait(0, 0)
        gather_start(0)
        idx_wait(1, 1)
        gather_start(1)

        emit(0, 0, swait_prev=False)
        emit(1, 1)

        def body(kk, carry):
            j0 = 2 + 4 * kk
            for u in range(4):
                emit(j0 + u, (2 + u) % _NBUF)
            return carry

        lax.fori_loop(0, (_NCHUNK - 5) // 4, body, 0)

        emit(_NCHUNK - 3, (_NCHUNK - 3) % _NBUF, idx_pf=False)
        emit(_NCHUNK - 2, (_NCHUNK - 2) % _NBUF, idx_pf=False, g_pf=False)
        emit(_NCHUNK - 1, (_NCHUNK - 1) % _NBUF, idx_pf=False, g_pf=False)
        scatter_wait((_NCHUNK - 1) % _NBUF)
        plsc.subcore_barrier()

        def wloop(i, carry):
            r0 = (s + i * _NS) * _WROWS
            pltpu.sync_copy(
                acc.at[pl.ds(r0, _WROWS)],
                out_hbm.at[c, pl.ds(r0, _WROWS)],
            )
            return carry

        lax.fori_loop(0, nz, wloop, 0)

    return k(x, src, dst)


_BLK = 1000


def _layer1_tc(x, p, w_root, w_nei, b):
    d = x.shape[1]
    h = w_root.shape[1]

    def body(x_ref, p0_ref, p1_ref, wr_ref, wn_ref, b_ref, o_ref):
        agg = p0_ref[...] + p1_ref[...]
        acc = jnp.dot(x_ref[...], wr_ref[...], preferred_element_type=jnp.float32)
        acc = acc + jnp.dot(agg, wn_ref[...], preferred_element_type=jnp.float32)
        o_ref[...] = jnp.maximum(acc + b_ref[...], 0.0)

    return pl.pallas_call(
        body,
        grid=(_N // _BLK,),
        in_specs=[
            pl.BlockSpec((_BLK, d), lambda i: (i, 0)),
            pl.BlockSpec((_BLK, d), lambda i: (i, 0)),
            pl.BlockSpec((_BLK, d), lambda i: (i, 0)),
            pl.BlockSpec((d, h), lambda i: (0, 0)),
            pl.BlockSpec((d, h), lambda i: (0, 0)),
            pl.BlockSpec((1, h), lambda i: (0, 0)),
        ],
        out_specs=pl.BlockSpec((_BLK, h), lambda i: (i, 0)),
        out_shape=jax.ShapeDtypeStruct((_N, h), jnp.float32),
    )(x, p[0], p[1], w_root, w_nei, b.reshape(1, h))


def _layer2_pool_tc(x, p, w_root, w_nei, b, batch):
    d = x.shape[1]
    h = w_root.shape[1]
    nblk = _N // _BLK

    def body(x_ref, p0_ref, p1_ref, wr_ref, wn_ref, b_ref, bat_ref, o_ref,
             acc_ref, cnt_ref):
        i = pl.program_id(0)
        agg = p0_ref[...] + p1_ref[...]
        hh = jnp.dot(x_ref[...], wr_ref[...], preferred_element_type=jnp.float32)
        hh = hh + jnp.dot(agg, wn_ref[...], preferred_element_type=jnp.float32)
        hh = jnp.maximum(hh + b_ref[...], 0.0)

        onehot = (bat_ref[...] ==
                  lax.broadcasted_iota(jnp.int32, (_BLK, _G), 1)
                  ).astype(jnp.float32)
        part = lax.dot_general(onehot, hh, (((0,), (0,)), ((), ())),
                               preferred_element_type=jnp.float32)
        ones = jnp.ones((_BLK, h), jnp.float32)
        pcnt = lax.dot_general(onehot, ones, (((0,), (0,)), ((), ())),
                               preferred_element_type=jnp.float32)

        @pl.when(i == 0)
        def _():
            acc_ref[...] = jnp.zeros_like(acc_ref)
            cnt_ref[...] = jnp.zeros_like(cnt_ref)

        acc_ref[...] += part
        cnt_ref[...] += pcnt

        @pl.when(i == nblk - 1)
        def _():
            o_ref[...] = acc_ref[...] / jnp.maximum(cnt_ref[...], 1.0)

    return pl.pallas_call(
        body,
        grid=(nblk,),
        in_specs=[
            pl.BlockSpec((_BLK, d), lambda i: (i, 0)),
            pl.BlockSpec((_BLK, d), lambda i: (i, 0)),
            pl.BlockSpec((_BLK, d), lambda i: (i, 0)),
            pl.BlockSpec((d, h), lambda i: (0, 0)),
            pl.BlockSpec((d, h), lambda i: (0, 0)),
            pl.BlockSpec((1, h), lambda i: (0, 0)),
            pl.BlockSpec((_BLK, 1), lambda i: (i, 0)),
        ],
        out_specs=pl.BlockSpec((_G, h), lambda i: (0, 0)),
        out_shape=jax.ShapeDtypeStruct((_G, h), jnp.float32),
        scratch_shapes=[
            pltpu.VMEM((_G, h), jnp.float32),
            pltpu.VMEM((_G, h), jnp.float32),
        ],
    )(x, p[0], p[1], w_root, w_nei, b.reshape(1, h), batch.reshape(_N, 1))


def kernel(x, edge_index, batch, W1_root, W1_nei, b1, W2_root, W2_nei, b2):
    src = edge_index[0].reshape(_NW, _NCHUNK, 1, _CHUNK)
    dst = edge_index[1].reshape(_NW, _NCHUNK, 1, _CHUNK)
    p1 = _segment_sum_sc(x, src, dst)
    h = _layer1_tc(x, p1, W1_root, W1_nei, b1)
    p2 = _segment_sum_sc(h, src, dst)
    return _layer2_pool_tc(h, p2, W2_root, W2_nei, b2, batch)
